# baseline (speedup 1.0000x reference)
import functools

import jax
import jax.numpy as jnp
from jax import lax
from jax.experimental import pallas as pl
from jax.experimental.pallas import tpu as pltpu
from jax.experimental.pallas import tpu_sc as plsc

NC = 2
NS = 16
L = 16
CHUNK = 128

_LN2 = 0.6931471805599453
_SQRT2 = 1.4142135623730951


def _log_f32(x):
    bits = plsc.bitcast(x, jnp.int32)
    e = (bits >> 23) - 127
    m = plsc.bitcast((bits & 0x007FFFFF) | 0x3F800000, jnp.float32)
    big = m > _SQRT2
    m = jnp.where(big, m * 0.5, m)
    e = (e + jnp.where(big, 1, 0)).astype(jnp.float32)
    z = (m - 1.0) / (m + 1.0)
    z2 = z * z
    p = 1.0 + z2 * (1.0 / 3.0 + z2 * (1.0 / 5.0 + z2 * (1.0 / 7.0 + z2 * (1.0 / 9.0))))
    return e * _LN2 + 2.0 * z * p


def _make_sc_kernel(B, D, S, interpret=False):
    DS = D * S
    NW = NC * NS
    PW = B // NW
    NCH = PW // CHUNK
    GROUPS = CHUNK // L
    PITCH = CHUNK + 1

    mesh = plsc.VectorSubcoreMesh(core_axis_name="c", subcore_axis_name="s",
                                  num_cores=NC, num_subcores=NS)

    @functools.partial(
        pl.kernel,
        out_type=jax.ShapeDtypeStruct((B,), jnp.float32),
        mesh=mesh,
        scratch_types=[
            pltpu.VMEM((NCH, CHUNK), jnp.int32),
            pltpu.VMEM((NCH, CHUNK), jnp.int32),
            pltpu.VMEM((NCH, CHUNK, DS), jnp.float32),
            pltpu.VMEM((NCH, CHUNK, DS), jnp.float32),
            pltpu.VMEM((DS * PITCH,), jnp.float32),
            pltpu.VMEM((DS * PITCH,), jnp.float32),
            pltpu.VMEM((PW,), jnp.float32),
            pltpu.SemaphoreType.DMA((NCH,)),
            pltpu.SemaphoreType.DMA((NCH,)),
        ],
        compiler_params=pltpu.CompilerParams(needs_layout_passes=False,
                                             use_tc_tiling_on_sc=False),
        interpret=interpret,
    )
    def sc_kernel(i_hbm, j_hbm, v_hbm, w_hbm, out_hbm,
                  iv, jv, vrows, wrows, vcol, wcol, outv, sem_v, sem_w):
        wid = lax.axis_index("s") * NC + lax.axis_index("c")
        for k in range(NCH):
            base = wid * PW + k * CHUNK
            pltpu.sync_copy(i_hbm.at[pl.ds(base, CHUNK)], iv.at[k])
            pltpu.sync_copy(j_hbm.at[pl.ds(base, CHUNK)], jv.at[k])

        vd = [pltpu.async_copy(v_hbm.at[iv.at[k]], vrows.at[k], sem_v.at[k])
              for k in range(NCH)]
        wd = [pltpu.async_copy(w_hbm.at[jv.at[k]], wrows.at[k], sem_w.at[k])
              for k in range(NCH)]

        iota = lax.iota(jnp.int32, L)
        for k in range(NCH):
            vd[k].wait()
            wd[k].wait()
            vr = vrows.at[k]
            wr = wrows.at[k]

            @pl.loop(0, CHUNK)
            def _tr(p):
                for q in range(S):
                    idx = (iota + q * L) * PITCH + p
                    plsc.store_scatter(vcol, [idx], vr[p, pl.ds(q * L, L)])
                    plsc.store_scatter(wcol, [idx], wr[p, pl.ds(q * L, L)])

            @pl.loop(0, GROUPS)
            def _group(g):
                total = jnp.zeros((L,), jnp.float32)
                for s0 in range(0, S, 2):
                    accs = [[jnp.zeros((L,), jnp.float32) for _ in range(S)]
                            for _ in range(2)]
                    for d in range(D):
                        wv = [wcol[pl.ds((d * S + s0 + s) * PITCH + g * L, L)]
                              for s in range(2)]
                        vv = [vcol[pl.ds((d * S + t) * PITCH + g * L, L)]
                              for t in range(S)]
                        for s in range(2):
                            for t in range(S):
                                accs[s][t] = accs[s][t] + wv[s] * vv[t]
                    for s in range(2):
                        for t in range(S):
                            total = total + jnp.exp(accs[s][t])
                outv[pl.ds(k * CHUNK + g * L, L)] = _log_f32(total)

        pltpu.sync_copy(outv, out_hbm.at[pl.ds(wid * PW, PW)])

    return sc_kernel


def kernel(IJ, _, V, W):
    B = IJ.shape[0]
    VOCAB, D, S = V.shape
    I2 = IJ[:, 0].astype(jnp.int32)
    J2 = IJ[:, 1].astype(jnp.int32)
    Vf = V.reshape(VOCAB, D * S)
    Wf = W.reshape(W.shape[0], D * S)
    sc = _make_sc_kernel(B, D, S)
    return sc(I2, J2, Vf, Wf)


# speedup vs baseline: 1.0238x; 1.0238x over previous
import functools

import jax
import jax.numpy as jnp
from jax import lax
from jax.experimental import pallas as pl
from jax.experimental.pallas import tpu as pltpu
from jax.experimental.pallas import tpu_sc as plsc

NC = 2
NS = 16
L = 16
CHUNK = 128

_LN2 = 0.6931471805599453
_SQRT2 = 1.4142135623730951


def _log_f32(x):
    bits = plsc.bitcast(x, jnp.int32)
    e = (bits >> 23) - 127
    m = plsc.bitcast((bits & 0x007FFFFF) | 0x3F800000, jnp.float32)
    big = m > _SQRT2
    m = jnp.where(big, m * 0.5, m)
    e = (e + jnp.where(big, 1, 0)).astype(jnp.float32)
    z = (m - 1.0) / (m + 1.0)
    z2 = z * z
    p = 1.0 + z2 * (1.0 / 3.0 + z2 * (1.0 / 5.0 + z2 * (1.0 / 7.0 + z2 * (1.0 / 9.0))))
    return e * _LN2 + 2.0 * z * p


def _make_sc_kernel(B, D, S, interpret=False):
    DS = D * S
    NW = NC * NS
    PW = B // NW
    NCH = PW // CHUNK
    GROUPS = CHUNK // L
    PITCH = CHUNK + 1

    mesh = plsc.VectorSubcoreMesh(core_axis_name="c", subcore_axis_name="s",
                                  num_cores=NC, num_subcores=NS)

    @functools.partial(
        pl.kernel,
        out_type=jax.ShapeDtypeStruct((B,), jnp.float32),
        mesh=mesh,
        scratch_types=[
            pltpu.VMEM((NCH, CHUNK), jnp.int32),
            pltpu.VMEM((NCH, CHUNK), jnp.int32),
            pltpu.VMEM((NCH, CHUNK, DS), jnp.float32),
            pltpu.VMEM((NCH, CHUNK, DS), jnp.float32),
            pltpu.VMEM((DS * PITCH,), jnp.float32),
            pltpu.VMEM((DS * PITCH,), jnp.float32),
            pltpu.VMEM((PW,), jnp.float32),
            pltpu.SemaphoreType.DMA((NCH,)),
            pltpu.SemaphoreType.DMA((NCH,)),
            pltpu.SemaphoreType.DMA((NCH,)),
            pltpu.SemaphoreType.DMA((NCH,)),
        ],
        compiler_params=pltpu.CompilerParams(needs_layout_passes=False,
                                             use_tc_tiling_on_sc=False),
        interpret=interpret,
    )
    def sc_kernel(i_hbm, j_hbm, v_hbm, w_hbm, out_hbm,
                  iv, jv, vrows, wrows, vcol, wcol, outv,
                  sem_v, sem_w, sem_iv, sem_jv):
        wid = lax.axis_index("s") * NC + lax.axis_index("c")
        idma = []
        for k in range(NCH):
            base = wid * PW + k * CHUNK
            idma.append((
                pltpu.async_copy(i_hbm.at[pl.ds(base, CHUNK)], iv.at[k],
                                 sem_iv.at[k]),
                pltpu.async_copy(j_hbm.at[pl.ds(base, CHUNK)], jv.at[k],
                                 sem_jv.at[k])))

        vd, wd = [], []
        for k in range(NCH):
            idma[k][0].wait()
            idma[k][1].wait()
            vd.append(pltpu.async_copy(v_hbm.at[iv.at[k]], vrows.at[k],
                                       sem_v.at[k]))
            wd.append(pltpu.async_copy(w_hbm.at[jv.at[k]], wrows.at[k],
                                       sem_w.at[k]))

        iota = lax.iota(jnp.int32, L)
        for k in range(NCH):
            vd[k].wait()
            wd[k].wait()
            vr = vrows.at[k]
            wr = wrows.at[k]

            @pl.loop(0, CHUNK)
            def _tr(p):
                for q in range(S):
                    idx = (iota + q * L) * PITCH + p
                    plsc.store_scatter(vcol, [idx], vr[p, pl.ds(q * L, L)])
                    plsc.store_scatter(wcol, [idx], wr[p, pl.ds(q * L, L)])

            @pl.loop(0, GROUPS)
            def _group(g):
                accs = [[jnp.zeros((L,), jnp.float32) for _ in range(S)]
                        for _ in range(S)]
                for d in range(D):
                    wv = [wcol[pl.ds((d * S + s) * PITCH + g * L, L)]
                          for s in range(S)]
                    vv = [vcol[pl.ds((d * S + t) * PITCH + g * L, L)]
                          for t in range(S)]
                    for s in range(S):
                        for t in range(S):
                            accs[s][t] = accs[s][t] + wv[s] * vv[t]
                total = jnp.zeros((L,), jnp.float32)
                for s in range(S):
                    for t in range(S):
                        total = total + jnp.exp(accs[s][t])
                outv[pl.ds(k * CHUNK + g * L, L)] = _log_f32(total)

        pltpu.sync_copy(outv, out_hbm.at[pl.ds(wid * PW, PW)])

    return sc_kernel


def kernel(IJ, _, V, W):
    B = IJ.shape[0]
    VOCAB, D, S = V.shape
    I2 = IJ[:, 0].astype(jnp.int32)
    J2 = IJ[:, 1].astype(jnp.int32)
    Vf = V.reshape(VOCAB, D * S)
    Wf = W.reshape(W.shape[0], D * S)
    sc = _make_sc_kernel(B, D, S)
    return sc(I2, J2, Vf, Wf)
